# baseline (speedup 1.0000x reference)
import functools

import jax
import jax.numpy as jnp
from jax import lax
from jax.experimental import pallas as pl
from jax.experimental.pallas import tpu as pltpu
from jax.experimental.pallas import tpu_sc as plsc

_N = 10000
_D = 128
_E = 320000
_EL = 100000

_NC = 2
_NS = 16
_NW = _NC * _NS

_K = 80
_EPW = _E // _NW
_CHUNKS = _EPW // _K

_ELP = 102400
_LPW = _ELP // _NW
_LCHUNKS = _LPW // _K

_NP = 10240
_STRIPE = _NP // _NS


def _lrelu(v):
    return jnp.where(v >= 0, v, 0.01 * v)


_sc_mesh = plsc.VectorSubcoreMesh(
    core_axis_name="c", subcore_axis_name="s",
    num_cores=_NC, num_subcores=_NS)
_sc_params = pltpu.CompilerParams(needs_layout_passes=False)


@functools.partial(
    pl.kernel,
    out_type=jax.ShapeDtypeStruct((_NC, _NP, _D), jnp.float32),
    mesh=_sc_mesh,
    scratch_types=(
        pltpu.VMEM((_EPW,), jnp.int32),
        pltpu.VMEM((_CHUNKS, _K), jnp.int32),
        pltpu.VMEM((_K, _D), jnp.float32),
        pltpu.VMEM((_K, _D), jnp.float32),
        pltpu.VMEM_SHARED((_NP, _D), jnp.float32),
        pltpu.SemaphoreType.DMA,
        pltpu.SemaphoreType.DMA,
        pltpu.SemaphoreType.DMA,
        pltpu.SemaphoreType.DMA,
    ),
    compiler_params=_sc_params)
def _scatter(h_hbm, src_hbm, dst_hbm, z128_hbm,
             acc_out, src_v, dst_v, rows0, rows1, acc_sh,
             sg0, sg1, ss0, ss1):
    c = lax.axis_index("c")
    s = lax.axis_index("s")
    w = c * _NS + s
    row0 = s * _STRIPE

    pltpu.sync_copy(z128_hbm.at[pl.ds(row0, _STRIPE)],
                    acc_sh.at[pl.ds(row0, _STRIPE)])

    pltpu.sync_copy(src_hbm.at[pl.ds(w * _EPW, _EPW)], src_v)
    pltpu.sync_copy(dst_hbm.at[w], dst_v)
    plsc.subcore_barrier()

    def fire_g(j, buf, sem):
        pltpu.async_copy(h_hbm.at[src_v.at[pl.ds(j * _K, _K)]], buf, sem)

    def drain_g(buf, sem):
        pltpu.make_async_copy(h_hbm.at[src_v.at[pl.ds(0, _K)]],
                              buf, sem).wait()

    def fire_s(j, buf, sem):
        pltpu.async_copy(buf, acc_sh.at[dst_v.at[j]], sem, add=True)

    def drain_s(buf, sem):
        pltpu.make_async_copy(buf, acc_sh.at[dst_v.at[0]], sem).wait()

    fire_g(0, rows0, sg0)
    fire_g(1, rows1, sg1)

    def body(i, _):
        j = 2 * i
        drain_g(rows0, sg0)
        fire_s(j, rows0, ss0)
        drain_g(rows1, sg1)
        fire_s(j + 1, rows1, ss1)
        drain_s(rows0, ss0)

        @pl.when(j + 2 < _CHUNKS)
        def _():
            fire_g(j + 2, rows0, sg0)
        drain_s(rows1, ss1)

        @pl.when(j + 3 < _CHUNKS)
        def _():
            fire_g(j + 3, rows1, sg1)
        return 0

    lax.fori_loop(0, _CHUNKS // 2, body, 0)
    drain_g(rows0, sg0)
    fire_s(_CHUNKS - 1, rows0, ss0)
    drain_s(rows0, ss0)
    plsc.subcore_barrier()

    pltpu.sync_copy(acc_sh.at[pl.ds(row0, _STRIPE)],
                    acc_out.at[c].at[pl.ds(row0, _STRIPE)])


@functools.partial(
    pl.kernel,
    out_type=jax.ShapeDtypeStruct((_NC, _NP, _D), jnp.float32),
    mesh=_sc_mesh,
    scratch_types=(
        pltpu.VMEM((_CHUNKS, _K), jnp.int32),
        pltpu.VMEM((_K, _D), jnp.float32),
        pltpu.VMEM_SHARED((_NP, _D), jnp.float32),
        pltpu.SemaphoreType.DMA,
    ),
    compiler_params=_sc_params)
def _count(dst_hbm, z128_hbm, ones_hbm, cnt_out, dst_v, ones_v, cnt_sh, sem):
    c = lax.axis_index("c")
    s = lax.axis_index("s")
    w = c * _NS + s
    row0 = s * _STRIPE

    pltpu.sync_copy(z128_hbm.at[pl.ds(row0, _STRIPE)],
                    cnt_sh.at[pl.ds(row0, _STRIPE)])
    pltpu.sync_copy(ones_hbm, ones_v)
    pltpu.sync_copy(dst_hbm.at[w], dst_v)
    plsc.subcore_barrier()

    def body(i, _):
        for b in range(5):
            pltpu.async_copy(ones_v, cnt_sh.at[dst_v.at[5 * i + b]],
                             sem, add=True)
        for b in range(5):
            pltpu.make_async_copy(ones_v, cnt_sh.at[dst_v.at[0]],
                                  sem).wait()
        return 0

    lax.fori_loop(0, _CHUNKS // 5, body, 0)
    plsc.subcore_barrier()

    pltpu.sync_copy(cnt_sh.at[pl.ds(row0, _STRIPE)],
                    cnt_out.at[c].at[pl.ds(row0, _STRIPE)])


_dot_mesh = plsc.VectorSubcoreMesh(
    core_axis_name="c", subcore_axis_name="s",
    num_cores=_NC, num_subcores=_NS)


@functools.partial(
    pl.kernel,
    out_type=jax.ShapeDtypeStruct((_ELP,), jnp.float32),
    mesh=_dot_mesh,
    scratch_types=(
        pltpu.VMEM((_LPW,), jnp.int32),
        pltpu.VMEM((_LPW,), jnp.int32),
        pltpu.VMEM((_K, _D), jnp.float32),
        pltpu.VMEM((_K, _D), jnp.float32),
        pltpu.VMEM((_K, _D), jnp.float32),
        pltpu.VMEM((_K, _D), jnp.float32),
        pltpu.VMEM((_LPW,), jnp.float32),
        pltpu.SemaphoreType.DMA,
        pltpu.SemaphoreType.DMA,
    ),
    compiler_params=_sc_params)
def _dot_kernel(h_hbm, ia_hbm, ib_hbm, out_hbm,
                ia_v, ib_v, rows_a0, rows_b0, rows_a1, rows_b1,
                out_v, sem0, sem1):
    c = lax.axis_index("c")
    s = lax.axis_index("s")
    w = c * _NS + s
    base = w * _LPW
    pltpu.sync_copy(ia_hbm.at[pl.ds(base, _LPW)], ia_v)
    pltpu.sync_copy(ib_hbm.at[pl.ds(base, _LPW)], ib_v)

    lanes = lax.iota(jnp.int32, 16)

    def fire(j, ra, rb, sem):
        pltpu.async_copy(h_hbm.at[ia_v.at[pl.ds(j * _K, _K)]], ra, sem)
        pltpu.async_copy(h_hbm.at[ib_v.at[pl.ds(j * _K, _K)]], rb, sem)

    def drain(ra, rb, sem):
        pltpu.make_async_copy(h_hbm.at[ia_v.at[pl.ds(0, _K)]], ra, sem).wait()
        pltpu.make_async_copy(h_hbm.at[ib_v.at[pl.ds(0, _K)]], rb, sem).wait()

    def compute(j, ra, rb):
        def group(g, _):
            rows_idx = g * 16 + lanes
            accs = [jnp.zeros((16,), jnp.float32) for _ in range(4)]
            for col in range(_D):
                colv = jnp.full((16,), col, jnp.int32)
                accs[col % 4] = accs[col % 4] + (
                    plsc.load_gather(ra, [rows_idx, colv]) *
                    plsc.load_gather(rb, [rows_idx, colv]))
            out_v[pl.ds(j * _K + g * 16, 16)] = (
                (accs[0] + accs[1]) + (accs[2] + accs[3]))
            return 0

        lax.fori_loop(0, _K // 16, group, 0)

    fire(0, rows_a0, rows_b0, sem0)

    def chunk(i, _):
        j = 2 * i
        fire(j + 1, rows_a1, rows_b1, sem1)
        drain(rows_a0, rows_b0, sem0)
        compute(j, rows_a0, rows_b0)

        @pl.when(j + 2 < _LCHUNKS)
        def _():
            fire(j + 2, rows_a0, rows_b0, sem0)
        drain(rows_a1, rows_b1, sem1)
        compute(j + 1, rows_a1, rows_b1)
        return 0

    lax.fori_loop(0, _LCHUNKS // 2, chunk, 0)
    pltpu.sync_copy(out_v, out_hbm.at[pl.ds(base, _LPW)])


def _dense_body(acc_ref, cnt_ref, h_ref, wn_ref, bn_ref, ws_ref, bs_ref,
                wu_ref, bu_ref, o_ref, *, act):
    acc = acc_ref[0, 0:_N, :] + acc_ref[1, 0:_N, :]
    cnt = cnt_ref[0, 0:_N, 0:1] + cnt_ref[1, 0:_N, 0:1]
    agg = acc / jnp.maximum(cnt, 1.0)
    neigh = jnp.dot(agg, wn_ref[...],
                    preferred_element_type=jnp.float32) + bn_ref[...]
    sf = jnp.dot(h_ref[...], ws_ref[...],
                 preferred_element_type=jnp.float32) + bs_ref[...]
    out = (jnp.dot(neigh, wu_ref[0:_D, :], preferred_element_type=jnp.float32)
           + jnp.dot(sf, wu_ref[_D:2 * _D, :],
                     preferred_element_type=jnp.float32)
           + bu_ref[...])
    o_ref[...] = _lrelu(out) if act else out


def _dense_layer(acc_p, cnt_p, h, Wn, bn, Ws, bs, Wu, bu, act):
    return pl.pallas_call(
        functools.partial(_dense_body, act=act),
        out_shape=jax.ShapeDtypeStruct((_N, _D), jnp.float32),
    )(acc_p, cnt_p, h, Wn, bn.reshape(1, _D), Ws, bs.reshape(1, _D),
      Wu, bu.reshape(1, _D))


def _lrelu_body(x_ref, o_ref):
    o_ref[...] = _lrelu(x_ref[...])


def _lrelu_call(x):
    return pl.pallas_call(
        _lrelu_body,
        out_shape=jax.ShapeDtypeStruct(x.shape, x.dtype),
    )(x)


def kernel(x, edge_index, edge_label_index,
           Wn1, bn1, Ws1, bs1, Wu1, bu1,
           Wn2, bn2, Ws2, bs2, Wu2, bu2):
    src = edge_index[0]
    dst3d = edge_index[1].reshape(_NW, _CHUNKS, _K)
    z128 = jnp.zeros((_NP, _D), jnp.float32)
    ones128 = jnp.ones((_K, _D), jnp.float32)

    pad = jnp.zeros((2, _ELP - _EL), jnp.int32)
    eli = jnp.concatenate([edge_label_index, pad], axis=1)

    h0 = _lrelu_call(x)
    cnt = _count(dst3d, z128, ones128)
    acc1 = _scatter(h0, src, dst3d, z128)
    g1 = _dense_layer(acc1, cnt, h0, Wn1, bn1, Ws1, bs1, Wu1, bu1, act=True)
    acc2 = _scatter(g1, src, dst3d, z128)
    h2 = _dense_layer(acc2, cnt, g1, Wn2, bn2, Ws2, bs2, Wu2, bu2, act=False)
    pred_pad = _dot_kernel(h2, eli[0], eli[1])
    return pred_pad[:_EL]


# speedup vs baseline: 1.2916x; 1.2916x over previous
import functools

import jax
import jax.numpy as jnp
from jax import lax
from jax.experimental import pallas as pl
from jax.experimental.pallas import tpu as pltpu
from jax.experimental.pallas import tpu_sc as plsc

_N = 10000
_D = 128
_E = 320000
_EL = 100000

_NC = 2
_NS = 16
_NW = _NC * _NS

_K = 80
_EPW = _E // _NW
_CHUNKS = _EPW // _K

_ELP = 102400
_LPW = _ELP // _NW
_LCHUNKS = _LPW // _K

_NP = 10240
_STRIPE = _NP // _NS


def _lrelu(v):
    return jnp.where(v >= 0, v, 0.01 * v)


_sc_mesh = plsc.VectorSubcoreMesh(
    core_axis_name="c", subcore_axis_name="s",
    num_cores=_NC, num_subcores=_NS)
_sc_params = pltpu.CompilerParams(needs_layout_passes=False)


@functools.partial(
    pl.kernel,
    out_type=jax.ShapeDtypeStruct((_NC, _NP, _D), jnp.float32),
    mesh=_sc_mesh,
    scratch_types=(
        pltpu.VMEM((_EPW,), jnp.int32),
        pltpu.VMEM((_CHUNKS, _K), jnp.int32),
        pltpu.VMEM((_K, _D), jnp.float32),
        pltpu.VMEM((_K, _D), jnp.float32),
        pltpu.VMEM_SHARED((_NP, _D), jnp.float32),
        pltpu.SemaphoreType.DMA,
        pltpu.SemaphoreType.DMA,
        pltpu.SemaphoreType.DMA,
        pltpu.SemaphoreType.DMA,
    ),
    compiler_params=_sc_params)
def _scatter(h_hbm, src_hbm, dst_hbm, z128_hbm,
             acc_out, src_v, dst_v, rows0, rows1, acc_sh,
             sg0, sg1, ss0, ss1):
    c = lax.axis_index("c")
    s = lax.axis_index("s")
    w = c * _NS + s
    row0 = s * _STRIPE

    pltpu.sync_copy(z128_hbm.at[pl.ds(row0, _STRIPE)],
                    acc_sh.at[pl.ds(row0, _STRIPE)])

    pltpu.sync_copy(src_hbm.at[pl.ds(w * _EPW, _EPW)], src_v)
    pltpu.sync_copy(dst_hbm.at[w], dst_v)
    plsc.subcore_barrier()

    def fire_g(j, buf, sem):
        pltpu.async_copy(h_hbm.at[src_v.at[pl.ds(j * _K, _K)]], buf, sem)

    def drain_g(buf, sem):
        pltpu.make_async_copy(h_hbm.at[src_v.at[pl.ds(0, _K)]],
                              buf, sem).wait()

    def fire_s(j, buf, sem):
        pltpu.async_copy(buf, acc_sh.at[dst_v.at[j]], sem, add=True)

    def drain_s(buf, sem):
        pltpu.make_async_copy(buf, acc_sh.at[dst_v.at[0]], sem).wait()

    fire_g(0, rows0, sg0)
    fire_g(1, rows1, sg1)

    def body(i, _):
        j = 2 * i
        drain_g(rows0, sg0)
        fire_s(j, rows0, ss0)
        drain_g(rows1, sg1)
        fire_s(j + 1, rows1, ss1)
        drain_s(rows0, ss0)

        @pl.when(j + 2 < _CHUNKS)
        def _():
            fire_g(j + 2, rows0, sg0)
        drain_s(rows1, ss1)

        @pl.when(j + 3 < _CHUNKS)
        def _():
            fire_g(j + 3, rows1, sg1)
        return 0

    lax.fori_loop(0, _CHUNKS // 2, body, 0)
    drain_g(rows0, sg0)
    fire_s(_CHUNKS - 1, rows0, ss0)
    drain_s(rows0, ss0)
    plsc.subcore_barrier()

    pltpu.sync_copy(acc_sh.at[pl.ds(row0, _STRIPE)],
                    acc_out.at[c].at[pl.ds(row0, _STRIPE)])


@functools.partial(
    pl.kernel,
    out_type=jax.ShapeDtypeStruct((_NC, _NP, _D), jnp.float32),
    mesh=_sc_mesh,
    scratch_types=(
        pltpu.VMEM((_CHUNKS, _K), jnp.int32),
        pltpu.VMEM((_K, _D), jnp.float32),
        pltpu.VMEM_SHARED((_NP, _D), jnp.float32),
        pltpu.SemaphoreType.DMA,
    ),
    compiler_params=_sc_params)
def _count(dst_hbm, z128_hbm, ones_hbm, cnt_out, dst_v, ones_v, cnt_sh, sem):
    c = lax.axis_index("c")
    s = lax.axis_index("s")
    w = c * _NS + s
    row0 = s * _STRIPE

    pltpu.sync_copy(z128_hbm.at[pl.ds(row0, _STRIPE)],
                    cnt_sh.at[pl.ds(row0, _STRIPE)])
    pltpu.sync_copy(ones_hbm, ones_v)
    pltpu.sync_copy(dst_hbm.at[w], dst_v)
    plsc.subcore_barrier()

    def body(i, _):
        for b in range(5):
            pltpu.async_copy(ones_v, cnt_sh.at[dst_v.at[5 * i + b]],
                             sem, add=True)
        for b in range(5):
            pltpu.make_async_copy(ones_v, cnt_sh.at[dst_v.at[0]],
                                  sem).wait()
        return 0

    lax.fori_loop(0, _CHUNKS // 5, body, 0)
    plsc.subcore_barrier()

    pltpu.sync_copy(cnt_sh.at[pl.ds(row0, _STRIPE)],
                    cnt_out.at[c].at[pl.ds(row0, _STRIPE)])


_dot_mesh = plsc.VectorSubcoreMesh(
    core_axis_name="c", subcore_axis_name="s",
    num_cores=_NC, num_subcores=_NS)


@functools.partial(
    pl.kernel,
    out_type=jax.ShapeDtypeStruct((_ELP,), jnp.float32),
    mesh=_dot_mesh,
    scratch_types=(
        pltpu.VMEM((_LPW,), jnp.int32),
        pltpu.VMEM((_LPW,), jnp.int32),
        pltpu.VMEM((_K, _D), jnp.float32),
        pltpu.VMEM((_K, _D), jnp.float32),
        pltpu.VMEM((_K, _D), jnp.float32),
        pltpu.VMEM((_K, _D), jnp.float32),
        pltpu.VMEM((_LPW,), jnp.float32),
        pltpu.SemaphoreType.DMA,
        pltpu.SemaphoreType.DMA,
    ),
    compiler_params=_sc_params)
def _dot_kernel(h_hbm, ia_hbm, ib_hbm, out_hbm,
                ia_v, ib_v, rows_a0, rows_b0, rows_a1, rows_b1,
                out_v, sem0, sem1):
    c = lax.axis_index("c")
    s = lax.axis_index("s")
    w = c * _NS + s
    base = w * _LPW
    pltpu.sync_copy(ia_hbm.at[pl.ds(base, _LPW)], ia_v)
    pltpu.sync_copy(ib_hbm.at[pl.ds(base, _LPW)], ib_v)

    lanes = lax.iota(jnp.int32, 16)

    def fire(j, ra, rb, sem):
        pltpu.async_copy(h_hbm.at[ia_v.at[pl.ds(j * _K, _K)]], ra, sem)
        pltpu.async_copy(h_hbm.at[ib_v.at[pl.ds(j * _K, _K)]], rb, sem)

    def drain(ra, rb, sem):
        pltpu.make_async_copy(h_hbm.at[ia_v.at[pl.ds(0, _K)]], ra, sem).wait()
        pltpu.make_async_copy(h_hbm.at[ib_v.at[pl.ds(0, _K)]], rb, sem).wait()

    def compute(j, ra, rb):
        def group(g, _):
            vec = jnp.zeros((16,), jnp.float32)
            for rr in range(16):
                r = g * 16 + rr
                prods = [ra[r, pl.ds(16 * k, 16)] * rb[r, pl.ds(16 * k, 16)]
                         for k in range(8)]
                p = ((prods[0] + prods[1]) + (prods[2] + prods[3])) + \
                    ((prods[4] + prods[5]) + (prods[6] + prods[7]))
                vec = jnp.where(lanes == rr, jnp.sum(p), vec)
            out_v[pl.ds(j * _K + g * 16, 16)] = vec
            return 0

        lax.fori_loop(0, _K // 16, group, 0)

    fire(0, rows_a0, rows_b0, sem0)

    def chunk(i, _):
        j = 2 * i
        fire(j + 1, rows_a1, rows_b1, sem1)
        drain(rows_a0, rows_b0, sem0)
        compute(j, rows_a0, rows_b0)

        @pl.when(j + 2 < _LCHUNKS)
        def _():
            fire(j + 2, rows_a0, rows_b0, sem0)
        drain(rows_a1, rows_b1, sem1)
        compute(j + 1, rows_a1, rows_b1)
        return 0

    lax.fori_loop(0, _LCHUNKS // 2, chunk, 0)
    pltpu.sync_copy(out_v, out_hbm.at[pl.ds(base, _LPW)])


def _dense_body(acc_ref, cnt_ref, h_ref, wn_ref, bn_ref, ws_ref, bs_ref,
                wu_ref, bu_ref, o_ref, *, act):
    acc = acc_ref[0, 0:_N, :] + acc_ref[1, 0:_N, :]
    cnt = cnt_ref[0, 0:_N, 0:1] + cnt_ref[1, 0:_N, 0:1]
    agg = acc / jnp.maximum(cnt, 1.0)
    neigh = jnp.dot(agg, wn_ref[...],
                    preferred_element_type=jnp.float32) + bn_ref[...]
    sf = jnp.dot(h_ref[...], ws_ref[...],
                 preferred_element_type=jnp.float32) + bs_ref[...]
    out = (jnp.dot(neigh, wu_ref[0:_D, :], preferred_element_type=jnp.float32)
           + jnp.dot(sf, wu_ref[_D:2 * _D, :],
                     preferred_element_type=jnp.float32)
           + bu_ref[...])
    o_ref[...] = _lrelu(out) if act else out


def _dense_layer(acc_p, cnt_p, h, Wn, bn, Ws, bs, Wu, bu, act):
    return pl.pallas_call(
        functools.partial(_dense_body, act=act),
        out_shape=jax.ShapeDtypeStruct((_N, _D), jnp.float32),
    )(acc_p, cnt_p, h, Wn, bn.reshape(1, _D), Ws, bs.reshape(1, _D),
      Wu, bu.reshape(1, _D))


def _lrelu_body(x_ref, o_ref):
    o_ref[...] = _lrelu(x_ref[...])


def _lrelu_call(x):
    return pl.pallas_call(
        _lrelu_body,
        out_shape=jax.ShapeDtypeStruct(x.shape, x.dtype),
    )(x)


def kernel(x, edge_index, edge_label_index,
           Wn1, bn1, Ws1, bs1, Wu1, bu1,
           Wn2, bn2, Ws2, bs2, Wu2, bu2):
    src = edge_index[0]
    dst3d = edge_index[1].reshape(_NW, _CHUNKS, _K)
    z128 = jnp.zeros((_NP, _D), jnp.float32)
    ones128 = jnp.ones((_K, _D), jnp.float32)

    pad = jnp.zeros((2, _ELP - _EL), jnp.int32)
    eli = jnp.concatenate([edge_label_index, pad], axis=1)

    h0 = _lrelu_call(x)
    cnt = _count(dst3d, z128, ones128)
    acc1 = _scatter(h0, src, dst3d, z128)
    g1 = _dense_layer(acc1, cnt, h0, Wn1, bn1, Ws1, bs1, Wu1, bu1, act=True)
    acc2 = _scatter(g1, src, dst3d, z128)
    h2 = _dense_layer(acc2, cnt, g1, Wn2, bn2, Ws2, bs2, Wu2, bu2, act=False)
    pred_pad = _dot_kernel(h2, eli[0], eli[1])
    return pred_pad[:_EL]
